# baseline (speedup 1.0000x reference)
import functools

import numpy as np
import jax
import jax.numpy as jnp
from jax import lax
from jax.experimental import pallas as pl
from jax.experimental.pallas import tpu as pltpu
from jax.experimental.pallas import tpu_sc as plsc

N = 10000
E = 320000
EPS = 1e-5

NC = 2
NS = 16
NW = NC * NS
EPT = E // NW
K = 96
EPP = 10080
NCH = EPP // K
DUMP = 10200
NP = 10240
RPT = NP // NS
DEGW = 16


def _sc_mesh():
    return plsc.VectorSubcoreMesh(core_axis_name="c", subcore_axis_name="s",
                                  num_cores=NC, num_subcores=NS)


def _deg_body(src_h, ones_h, z_h, out_h, src_v, ones_v, acc, sem):
    cid = lax.axis_index("c")
    sid = lax.axis_index("s")
    wid = sid * NC + cid
    pltpu.sync_copy(src_h.at[wid], src_v)
    pltpu.sync_copy(ones_h, ones_v)
    pltpu.sync_copy(z_h.at[pl.ds(sid * RPT, RPT)], acc.at[pl.ds(sid * RPT, RPT)])
    plsc.subcore_barrier()

    def body(j, carry):
        pltpu.sync_copy(ones_v, acc.at[src_v.at[j]], add=True)
        return carry

    lax.fori_loop(0, NCH, body, 0)
    plsc.subcore_barrier()
    pltpu.sync_copy(acc.at[pl.ds(sid * RPT, RPT)],
                    out_h.at[cid, pl.ds(sid * RPT, RPT)])


def _deg_call(srcw, ones_h, z_h):
    return pl.kernel(
        _deg_body,
        out_type=jax.ShapeDtypeStruct((NC, NP, DEGW), jnp.float32),
        mesh=_sc_mesh(),
        scratch_types=[
            pltpu.VMEM((NCH, K), jnp.int32),
            pltpu.VMEM((K, DEGW), jnp.float32),
            pltpu.VMEM_SHARED((NP, DEGW), jnp.float32),
            pltpu.SemaphoreType.DMA,
        ],
        compiler_params=pltpu.CompilerParams(use_tc_tiling_on_sc=False),
    )(srcw, ones_h, z_h)


def _gs_body(g_h, src_h, dst_h, z_h, out_h, src_v, dst_v, rows2, acc, gsem):
    cid = lax.axis_index("c")
    sid = lax.axis_index("s")
    wid = sid * NC + cid
    pltpu.sync_copy(src_h.at[wid], src_v)
    pltpu.sync_copy(dst_h.at[wid], dst_v)
    pltpu.sync_copy(z_h.at[pl.ds(sid * RPT, RPT)], acc.at[pl.ds(sid * RPT, RPT)])
    plsc.subcore_barrier()

    pltpu.async_copy(g_h.at[src_v.at[0]], rows2.at[0], gsem.at[0])

    def body(j, carry):
        p = lax.rem(j, 2)

        @pl.when(j + 1 < NCH)
        def _start_next():
            pltpu.async_copy(g_h.at[src_v.at[j + 1]], rows2.at[1 - p],
                             gsem.at[1 - p])

        pltpu.make_async_copy(g_h.at[src_v.at[j]], rows2.at[p],
                              gsem.at[p]).wait()
        pltpu.sync_copy(rows2.at[p], acc.at[dst_v.at[j]], add=True)
        return carry

    lax.fori_loop(0, NCH, body, 0)
    plsc.subcore_barrier()
    pltpu.sync_copy(acc.at[pl.ds(sid * RPT, RPT)],
                    out_h.at[cid, pl.ds(sid * RPT, RPT)])


def _gs_call(g, srcw, dstw, z_h):
    c = g.shape[1]
    return pl.kernel(
        _gs_body,
        out_type=jax.ShapeDtypeStruct((NC, NP, c), jnp.float32),
        mesh=_sc_mesh(),
        scratch_types=[
            pltpu.VMEM((NCH, K), jnp.int32),
            pltpu.VMEM((NCH, K), jnp.int32),
            pltpu.VMEM((2, K, c), jnp.float32),
            pltpu.VMEM_SHARED((NP, c), jnp.float32),
            pltpu.SemaphoreType.DMA((2,)),
        ],
        compiler_params=pltpu.CompilerParams(use_tc_tiling_on_sc=False),
    )(g, srcw, dstw, z_h)


def _bn(h, g, b):
    m = jnp.mean(h, axis=0, keepdims=True)
    v = jnp.mean((h - m) ** 2, axis=0, keepdims=True)
    return (h - m) * lax.rsqrt(v + EPS) * g + b


def _dot(a, b):
    return jnp.dot(a, b, preferred_element_type=jnp.float32)


def _b_body(x_r, degp_r, aw1_r, ab1_r, aw2_r, ab2_r, e4_r,
            fmw_r, fmb_r, bmg_r, bmb_r, faw_r, fab_r, bag_r, bab_r,
            dis_o, h0_o, g1_o, ixm_o, ixa_o):
    x = x_r[...]
    deg = degp_r[0, :N, 0:1] + degp_r[1, :N, 0:1]
    dis = jnp.where(deg > 0.0, lax.rsqrt(jnp.maximum(deg, 1e-30)), 0.0)
    dis_o[...] = dis
    xm = jnp.mean(x, axis=0, keepdims=True)
    hid = jnp.maximum(_dot(xm, aw1_r[...]) + ab1_r[...], 0.0)
    raw = _dot(hid, aw2_r[...]) + ab2_r[...]
    raw = raw - jnp.max(raw, axis=1, keepdims=True)
    er = jnp.exp(raw)
    dw = er / jnp.sum(er, axis=1, keepdims=True)
    dwc = _dot(dw, e4_r[...])
    h0 = x * dwc
    h0_o[...] = h0
    g1_o[...] = h0 * dis
    ixm_o[...] = _bn(jnp.maximum(_dot(x, fmw_r[...]) + fmb_r[...], 0.0),
                     bmg_r[...], bmb_r[...])
    ixa_o[...] = _bn(jnp.maximum(_dot(x, faw_r[...]) + fab_r[...], 0.0),
                     bag_r[...], bab_r[...])


def _d1_body(h0_r, sp_r, dis_r, w0_r, w1_r, b_r, bg_r, bb_r, h1_o, g2_o):
    dis = dis_r[...]
    t = (sp_r[0, :N] + sp_r[1, :N]) * (-dis)
    u = _dot(h0_r[...], w0_r[...]) + _dot(t, w1_r[...]) + b_r[...]
    h1 = jnp.maximum(_bn(u, bg_r[...], bb_r[...]), 0.0)
    h1_o[...] = h1
    g2_o[...] = h1 * dis


def _d2_body(h1_r, sp_r, dis_r, w0_r, w1_r, b_r, bg_r, bb_r, ixm_r, a_r,
             h2_o, g3_o):
    dis = dis_r[...]
    t = (sp_r[0, :N] + sp_r[1, :N]) * (-dis)
    u = _dot(h1_r[...], w0_r[...]) + _dot(t, w1_r[...]) + b_r[...]
    hm = jnp.maximum(_bn(u, bg_r[...], bb_r[...]), 0.0)
    a = a_r[0, 0]
    h2 = jnp.maximum((1.0 - a) * hm + a * ixm_r[...], 0.0)
    h2_o[...] = h2
    g3_o[...] = h2 * dis


def _d3_body(h2_r, sp_r, dis_r, w0_r, w1_r, b_r, bg_r, bb_r, ixa_r, a_r,
             out_o):
    dis = dis_r[...]
    t = (sp_r[0, :N] + sp_r[1, :N]) * (-dis)
    u = _dot(h2_r[...], w0_r[...]) + _dot(t, w1_r[...]) + b_r[...]
    h3 = _bn(u, bg_r[...], bb_r[...])
    a = a_r[0, 0]
    out_o[...] = jnp.maximum((1.0 - a) * h3 + a * ixa_r[...], 0.0)


def _sds(shape):
    return jax.ShapeDtypeStruct(shape, jnp.float32)


_E4 = np.zeros((128, 128), np.float32)
for _i in range(4):
    _E4[_i, 32 * _i:32 * (_i + 1)] = 1.0


def kernel(x, edge_index, params):
    p = params
    src = edge_index[0].reshape(NW, EPT)
    dst = edge_index[1].reshape(NW, EPT)
    padz = jnp.zeros((NW, EPP - EPT), jnp.int32)
    padd = jnp.full((NW, EPP - EPT), DUMP, jnp.int32)
    srcw = jnp.concatenate([src, padz], axis=1).reshape(NW, NCH, K)
    dstw = jnp.concatenate([dst, padd], axis=1).reshape(NW, NCH, K)
    srcw_deg = jnp.concatenate([src, padd], axis=1).reshape(NW, NCH, K)

    onesb = jnp.ones((K, DEGW), jnp.float32)
    zdeg = jnp.zeros((NP, DEGW), jnp.float32)
    z128 = jnp.zeros((NP, 128), jnp.float32)
    z64 = jnp.zeros((NP, 64), jnp.float32)

    degp = _deg_call(srcw_deg, onesb, zdeg)

    aw2p = jnp.zeros((128, 128), jnp.float32).at[:, :4].set(p['att_W2'])
    ab2p = jnp.full((1, 128), -1e30, jnp.float32).at[0, :4].set(p['att_b2'])
    e4 = jnp.asarray(_E4)

    r = lambda a: a.reshape(1, -1)
    dis, h0, g1, ixm, ixa = pl.pallas_call(
        _b_body,
        out_shape=[_sds((N, 1)), _sds((N, 128)), _sds((N, 128)),
                   _sds((N, 64)), _sds((N, 32))],
    )(x, degp, p['att_W1'], r(p['att_b1']), aw2p, ab2p, e4,
      p['fcm_W'], r(p['fcm_b']), r(p['bnm_g']), r(p['bnm_b']),
      p['fca_W'], r(p['fca_b']), r(p['bna_g']), r(p['bna_b']))

    s1p = _gs_call(g1, srcw, dstw, z128)
    h1, g2 = pl.pallas_call(
        _d1_body,
        out_shape=[_sds((N, 128)), _sds((N, 128))],
    )(h0, s1p, dis, p['c1_W0'], p['c1_W1'], r(p['c1_b']),
      r(p['bn1_g']), r(p['bn1_b']))

    s2p = _gs_call(g2, srcw, dstw, z128)
    h2, g3 = pl.pallas_call(
        _d2_body,
        out_shape=[_sds((N, 64)), _sds((N, 64))],
    )(h1, s2p, dis, p['c2_W0'], p['c2_W1'], r(p['c2_b']),
      r(p['bn2_g']), r(p['bn2_b']), ixm, p['alpha_main'].reshape(1, 1))

    s3p = _gs_call(g3, srcw, dstw, z64)
    out = pl.pallas_call(
        _d3_body,
        out_shape=_sds((N, 32)),
    )(h2, s3p, dis, p['c3_W0'], p['c3_W1'], r(p['c3_b']),
      r(p['bn3_g']), r(p['bn3_b']), ixa, p['alpha_aux'].reshape(1, 1))
    return out


# speedup vs baseline: 1.4760x; 1.4760x over previous
import functools

import numpy as np
import jax
import jax.numpy as jnp
from jax import lax
from jax.experimental import pallas as pl
from jax.experimental.pallas import tpu as pltpu
from jax.experimental.pallas import tpu_sc as plsc

N = 10000
E = 320000
EPS = 1e-5

NC = 2
NS = 16
NW = NC * NS
EPT = E // NW
K = 80
NCH = EPT // K
NP = 10240
RPT = NP // NS
DEGW = 16


def _sc_mesh():
    return plsc.VectorSubcoreMesh(core_axis_name="c", subcore_axis_name="s",
                                  num_cores=NC, num_subcores=NS)


def _deg_body(src_h, ones_h, z_h, out_h, src_v, ones_v, acc, sem):
    cid = lax.axis_index("c")
    sid = lax.axis_index("s")
    wid = sid * NC + cid
    pltpu.sync_copy(src_h.at[wid], src_v)
    pltpu.sync_copy(ones_h, ones_v)
    pltpu.sync_copy(z_h.at[pl.ds(sid * RPT, RPT)], acc.at[pl.ds(sid * RPT, RPT)])
    plsc.subcore_barrier()

    def body(j, carry):
        pltpu.async_copy(ones_v, acc.at[src_v.at[j]], sem, add=True)
        return carry

    lax.fori_loop(0, NCH, body, 0)

    def drain(j, carry):
        pltpu.make_async_copy(ones_v, acc.at[src_v.at[j]], sem).wait()
        return carry

    lax.fori_loop(0, NCH, drain, 0)
    plsc.subcore_barrier()
    pltpu.sync_copy(acc.at[pl.ds(sid * RPT, RPT)],
                    out_h.at[cid, pl.ds(sid * RPT, RPT)])


def _deg_call(srcw, ones_h, z_h):
    return pl.kernel(
        _deg_body,
        out_type=jax.ShapeDtypeStruct((NC, NP, DEGW), jnp.float32),
        mesh=_sc_mesh(),
        scratch_types=[
            pltpu.VMEM((NCH, K), jnp.int32),
            pltpu.VMEM((K, DEGW), jnp.float32),
            pltpu.VMEM_SHARED((NP, DEGW), jnp.float32),
            pltpu.SemaphoreType.DMA,
        ],
        compiler_params=pltpu.CompilerParams(use_tc_tiling_on_sc=False),
    )(srcw, ones_h, z_h)


def _gs_body(g_h, src_h, dst_h, z_h, out_h, src_v, dst_v, rows2, acc, gsem):
    cid = lax.axis_index("c")
    sid = lax.axis_index("s")
    wid = sid * NC + cid
    pltpu.sync_copy(src_h.at[wid], src_v)
    pltpu.sync_copy(dst_h.at[wid], dst_v)
    pltpu.sync_copy(z_h.at[pl.ds(sid * RPT, RPT)], acc.at[pl.ds(sid * RPT, RPT)])
    plsc.subcore_barrier()

    pltpu.async_copy(g_h.at[src_v.at[0]], rows2.at[0], gsem.at[0])

    def body(j, carry):
        p = lax.rem(j, 2)

        @pl.when(j + 1 < NCH)
        def _start_next():
            pltpu.async_copy(g_h.at[src_v.at[j + 1]], rows2.at[1 - p],
                             gsem.at[1 - p])

        pltpu.make_async_copy(g_h.at[src_v.at[j]], rows2.at[p],
                              gsem.at[p]).wait()
        pltpu.sync_copy(rows2.at[p], acc.at[dst_v.at[j]], add=True)
        return carry

    lax.fori_loop(0, NCH, body, 0)
    plsc.subcore_barrier()
    pltpu.sync_copy(acc.at[pl.ds(sid * RPT, RPT)],
                    out_h.at[cid, pl.ds(sid * RPT, RPT)])


def _gs_call(g, srcw, dstw, z_h):
    c = g.shape[1]
    return pl.kernel(
        _gs_body,
        out_type=jax.ShapeDtypeStruct((NC, NP, c), jnp.float32),
        mesh=_sc_mesh(),
        scratch_types=[
            pltpu.VMEM((NCH, K), jnp.int32),
            pltpu.VMEM((NCH, K), jnp.int32),
            pltpu.VMEM((2, K, c), jnp.float32),
            pltpu.VMEM_SHARED((NP, c), jnp.float32),
            pltpu.SemaphoreType.DMA((2,)),
        ],
        compiler_params=pltpu.CompilerParams(use_tc_tiling_on_sc=False),
    )(g, srcw, dstw, z_h)


def _bn(h, g, b):
    m = jnp.mean(h, axis=0, keepdims=True)
    v = jnp.mean((h - m) ** 2, axis=0, keepdims=True)
    return (h - m) * lax.rsqrt(v + EPS) * g + b


def _dot(a, b):
    return jnp.dot(a, b, preferred_element_type=jnp.float32)


def _b_body(x_r, degp_r, aw1_r, ab1_r, aw2_r, ab2_r, e4_r,
            fmw_r, fmb_r, bmg_r, bmb_r, faw_r, fab_r, bag_r, bab_r,
            dis_o, h0_o, g1_o, ixm_o, ixa_o):
    x = x_r[...]
    deg = degp_r[0, :N, 0:1] + degp_r[1, :N, 0:1]
    dis = jnp.where(deg > 0.0, lax.rsqrt(jnp.maximum(deg, 1e-30)), 0.0)
    dis_o[...] = dis
    xm = jnp.mean(x, axis=0, keepdims=True)
    hid = jnp.maximum(_dot(xm, aw1_r[...]) + ab1_r[...], 0.0)
    raw = _dot(hid, aw2_r[...]) + ab2_r[...]
    raw = raw - jnp.max(raw, axis=1, keepdims=True)
    er = jnp.exp(raw)
    dw = er / jnp.sum(er, axis=1, keepdims=True)
    dwc = _dot(dw, e4_r[...])
    h0 = x * dwc
    h0_o[...] = h0
    g1_o[...] = h0 * dis
    ixm_o[...] = _bn(jnp.maximum(_dot(x, fmw_r[...]) + fmb_r[...], 0.0),
                     bmg_r[...], bmb_r[...])
    ixa_o[...] = _bn(jnp.maximum(_dot(x, faw_r[...]) + fab_r[...], 0.0),
                     bag_r[...], bab_r[...])


def _d1_body(h0_r, sp_r, dis_r, w0_r, w1_r, b_r, bg_r, bb_r, h1_o, g2_o):
    dis = dis_r[...]
    t = (sp_r[0, :N] + sp_r[1, :N]) * (-dis)
    u = _dot(h0_r[...], w0_r[...]) + _dot(t, w1_r[...]) + b_r[...]
    h1 = jnp.maximum(_bn(u, bg_r[...], bb_r[...]), 0.0)
    h1_o[...] = h1
    g2_o[...] = h1 * dis


def _d2_body(h1_r, sp_r, dis_r, w0_r, w1_r, b_r, bg_r, bb_r, ixm_r, a_r,
             h2_o, g3_o):
    dis = dis_r[...]
    t = (sp_r[0, :N] + sp_r[1, :N]) * (-dis)
    u = _dot(h1_r[...], w0_r[...]) + _dot(t, w1_r[...]) + b_r[...]
    hm = jnp.maximum(_bn(u, bg_r[...], bb_r[...]), 0.0)
    a = a_r[0, 0]
    h2 = jnp.maximum((1.0 - a) * hm + a * ixm_r[...], 0.0)
    h2_o[...] = h2
    g3_o[...] = h2 * dis


def _d3_body(h2_r, sp_r, dis_r, w0_r, w1_r, b_r, bg_r, bb_r, ixa_r, a_r,
             out_o):
    dis = dis_r[...]
    t = (sp_r[0, :N] + sp_r[1, :N]) * (-dis)
    u = _dot(h2_r[...], w0_r[...]) + _dot(t, w1_r[...]) + b_r[...]
    h3 = _bn(u, bg_r[...], bb_r[...])
    a = a_r[0, 0]
    out_o[...] = jnp.maximum((1.0 - a) * h3 + a * ixa_r[...], 0.0)


def _sds(shape):
    return jax.ShapeDtypeStruct(shape, jnp.float32)


_E4 = np.zeros((128, 128), np.float32)
for _i in range(4):
    _E4[_i, 32 * _i:32 * (_i + 1)] = 1.0


def kernel(x, edge_index, params):
    p = params
    src = edge_index[0]
    dst = edge_index[1]
    srcw = src.reshape(NW, NCH, K)
    dstw = dst.reshape(NW, NCH, K)

    onesb = jnp.ones((K, DEGW), jnp.float32)
    zdeg = jnp.zeros((NP, DEGW), jnp.float32)
    z128 = jnp.zeros((NP, 128), jnp.float32)
    z64 = jnp.zeros((NP, 64), jnp.float32)

    degp = _deg_call(srcw, onesb, zdeg)

    aw2p = jnp.zeros((128, 128), jnp.float32).at[:, :4].set(p['att_W2'])
    ab2p = jnp.full((1, 128), -1e30, jnp.float32).at[0, :4].set(p['att_b2'])
    e4 = jnp.asarray(_E4)

    r = lambda a: a.reshape(1, -1)
    dis, h0, g1, ixm, ixa = pl.pallas_call(
        _b_body,
        out_shape=[_sds((N, 1)), _sds((N, 128)), _sds((N, 128)),
                   _sds((N, 64)), _sds((N, 32))],
    )(x, degp, p['att_W1'], r(p['att_b1']), aw2p, ab2p, e4,
      p['fcm_W'], r(p['fcm_b']), r(p['bnm_g']), r(p['bnm_b']),
      p['fca_W'], r(p['fca_b']), r(p['bna_g']), r(p['bna_b']))

    s1p = _gs_call(g1, srcw, dstw, z128)
    h1, g2 = pl.pallas_call(
        _d1_body,
        out_shape=[_sds((N, 128)), _sds((N, 128))],
    )(h0, s1p, dis, p['c1_W0'], p['c1_W1'], r(p['c1_b']),
      r(p['bn1_g']), r(p['bn1_b']))

    s2p = _gs_call(g2, srcw, dstw, z128)
    h2, g3 = pl.pallas_call(
        _d2_body,
        out_shape=[_sds((N, 64)), _sds((N, 64))],
    )(h1, s2p, dis, p['c2_W0'], p['c2_W1'], r(p['c2_b']),
      r(p['bn2_g']), r(p['bn2_b']), ixm, p['alpha_main'].reshape(1, 1))

    s3p = _gs_call(g3, srcw, dstw, z64)
    out = pl.pallas_call(
        _d3_body,
        out_shape=_sds((N, 32)),
    )(h2, s3p, dis, p['c3_W0'], p['c3_W1'], r(p['c3_b']),
      r(p['bn3_g']), r(p['bn3_b']), ixa, p['alpha_aux'].reshape(1, 1))
    return out


# speedup vs baseline: 1.4850x; 1.0061x over previous
import functools

import numpy as np
import jax
import jax.numpy as jnp
from jax import lax
from jax.experimental import pallas as pl
from jax.experimental.pallas import tpu as pltpu
from jax.experimental.pallas import tpu_sc as plsc

N = 10000
E = 320000
EPS = 1e-5

NC = 2
NS = 16
NW = NC * NS
EPT = E // NW
K = 80
NCH = EPT // K
NP = 10240
RPT = NP // NS
DEGW = 16


def _sc_mesh():
    return plsc.VectorSubcoreMesh(core_axis_name="c", subcore_axis_name="s",
                                  num_cores=NC, num_subcores=NS)


def _deg_body(src_h, ones_h, z_h, out_h, src_v, ones_v, acc, sem):
    cid = lax.axis_index("c")
    sid = lax.axis_index("s")
    wid = sid * NC + cid
    pltpu.sync_copy(src_h.at[wid], src_v)
    pltpu.sync_copy(ones_h, ones_v)
    pltpu.sync_copy(z_h.at[pl.ds(sid * RPT, RPT)], acc.at[pl.ds(sid * RPT, RPT)])
    plsc.subcore_barrier()

    def body(j, carry):
        pltpu.async_copy(ones_v, acc.at[src_v.at[j]], sem, add=True)
        return carry

    lax.fori_loop(0, NCH, body, 0)

    def drain(j, carry):
        pltpu.make_async_copy(ones_v, acc.at[src_v.at[j]], sem).wait()
        return carry

    lax.fori_loop(0, NCH, drain, 0)
    plsc.subcore_barrier()
    pltpu.sync_copy(acc.at[pl.ds(sid * RPT, RPT)],
                    out_h.at[cid, pl.ds(sid * RPT, RPT)])


def _deg_call(srcw, ones_h, z_h):
    return pl.kernel(
        _deg_body,
        out_type=jax.ShapeDtypeStruct((NC, NP, DEGW), jnp.float32),
        mesh=_sc_mesh(),
        scratch_types=[
            pltpu.VMEM((NCH, K), jnp.int32),
            pltpu.VMEM((K, DEGW), jnp.float32),
            pltpu.VMEM_SHARED((NP, DEGW), jnp.float32),
            pltpu.SemaphoreType.DMA,
        ],
        compiler_params=pltpu.CompilerParams(use_tc_tiling_on_sc=False),
    )(srcw, ones_h, z_h)


def _gs_body(g_h, src_h, dst_h, z_h, out_h, src_v, dst_v, rows2, acc, gsem):
    cid = lax.axis_index("c")
    sid = lax.axis_index("s")
    wid = sid * NC + cid
    pltpu.sync_copy(src_h.at[wid], src_v)
    pltpu.sync_copy(dst_h.at[wid], dst_v)
    pltpu.sync_copy(z_h.at[pl.ds(sid * RPT, RPT)], acc.at[pl.ds(sid * RPT, RPT)])
    plsc.subcore_barrier()

    pltpu.async_copy(g_h.at[src_v.at[0]], rows2.at[0], gsem.at[0])

    def body(j, carry):
        p = lax.rem(j, 2)

        @pl.when(j + 1 < NCH)
        def _start_next():
            pltpu.async_copy(g_h.at[src_v.at[j + 1]], rows2.at[1 - p],
                             gsem.at[1 - p])

        pltpu.make_async_copy(g_h.at[src_v.at[j]], rows2.at[p],
                              gsem.at[p]).wait()
        pltpu.sync_copy(rows2.at[p], acc.at[dst_v.at[j]], add=True)
        return carry

    lax.fori_loop(0, NCH, body, 0)
    plsc.subcore_barrier()
    pltpu.sync_copy(acc.at[pl.ds(sid * RPT, RPT)],
                    out_h.at[cid, pl.ds(sid * RPT, RPT)])


def _gs_call(g, srcw, dstw, z_h):
    c = g.shape[1]
    return pl.kernel(
        _gs_body,
        out_type=jax.ShapeDtypeStruct((NC, NP, c), jnp.float32),
        mesh=_sc_mesh(),
        scratch_types=[
            pltpu.VMEM((NCH, K), jnp.int32),
            pltpu.VMEM((NCH, K), jnp.int32),
            pltpu.VMEM((2, K, c), jnp.float32),
            pltpu.VMEM_SHARED((NP, c), jnp.float32),
            pltpu.SemaphoreType.DMA((2,)),
        ],
        compiler_params=pltpu.CompilerParams(use_tc_tiling_on_sc=False),
    )(g, srcw, dstw, z_h)


def _bn(h, g, b):
    m = jnp.mean(h, axis=0, keepdims=True)
    v = jnp.mean((h - m) ** 2, axis=0, keepdims=True)
    return (h - m) * lax.rsqrt(v + EPS) * g + b


def _dot(a, b):
    return jnp.dot(a, b, preferred_element_type=jnp.float32)


def _b0_body(x_r, aw1_r, ab1_r, aw2_r, ab2_r, e4_r,
             fmw_r, fmb_r, bmg_r, bmb_r, faw_r, fab_r, bag_r, bab_r,
             h0_o, ixm_o, ixa_o):
    x = x_r[...]
    xm = jnp.mean(x, axis=0, keepdims=True)
    hid = jnp.maximum(_dot(xm, aw1_r[...]) + ab1_r[...], 0.0)
    raw = _dot(hid, aw2_r[...]) + ab2_r[...]
    raw = raw - jnp.max(raw, axis=1, keepdims=True)
    er = jnp.exp(raw)
    dw = er / jnp.sum(er, axis=1, keepdims=True)
    dwc = _dot(dw, e4_r[...])
    h0 = x * dwc
    h0_o[...] = h0
    ixm_o[...] = _bn(jnp.maximum(_dot(x, fmw_r[...]) + fmb_r[...], 0.0),
                     bmg_r[...], bmb_r[...])
    ixa_o[...] = _bn(jnp.maximum(_dot(x, faw_r[...]) + fab_r[...], 0.0),
                     bag_r[...], bab_r[...])


def _b1_body(h0_r, degp_r, dis_o, g1_o):
    deg = degp_r[0, :N, 0:1] + degp_r[1, :N, 0:1]
    dis = jnp.where(deg > 0.0, lax.rsqrt(jnp.maximum(deg, 1e-30)), 0.0)
    dis_o[...] = dis
    g1_o[...] = h0_r[...] * dis


def _d1_body(h0_r, sp_r, dis_r, w0_r, w1_r, b_r, bg_r, bb_r, h1_o, g2_o):
    dis = dis_r[...]
    t = (sp_r[0, :N] + sp_r[1, :N]) * (-dis)
    u = _dot(h0_r[...], w0_r[...]) + _dot(t, w1_r[...]) + b_r[...]
    h1 = jnp.maximum(_bn(u, bg_r[...], bb_r[...]), 0.0)
    h1_o[...] = h1
    g2_o[...] = h1 * dis


def _d2_body(h1_r, sp_r, dis_r, w0_r, w1_r, b_r, bg_r, bb_r, ixm_r, a_r,
             h2_o, g3_o):
    dis = dis_r[...]
    t = (sp_r[0, :N] + sp_r[1, :N]) * (-dis)
    u = _dot(h1_r[...], w0_r[...]) + _dot(t, w1_r[...]) + b_r[...]
    hm = jnp.maximum(_bn(u, bg_r[...], bb_r[...]), 0.0)
    a = a_r[0, 0]
    h2 = jnp.maximum((1.0 - a) * hm + a * ixm_r[...], 0.0)
    h2_o[...] = h2
    g3_o[...] = h2 * dis


def _d3_body(h2_r, sp_r, dis_r, w0_r, w1_r, b_r, bg_r, bb_r, ixa_r, a_r,
             out_o):
    dis = dis_r[...]
    t = (sp_r[0, :N] + sp_r[1, :N]) * (-dis)
    u = _dot(h2_r[...], w0_r[...]) + _dot(t, w1_r[...]) + b_r[...]
    h3 = _bn(u, bg_r[...], bb_r[...])
    a = a_r[0, 0]
    out_o[...] = jnp.maximum((1.0 - a) * h3 + a * ixa_r[...], 0.0)


def _sds(shape):
    return jax.ShapeDtypeStruct(shape, jnp.float32)


_E4 = np.zeros((128, 128), np.float32)
for _i in range(4):
    _E4[_i, 32 * _i:32 * (_i + 1)] = 1.0


def kernel(x, edge_index, params):
    p = params
    src = edge_index[0]
    dst = edge_index[1]
    srcw = src.reshape(NW, NCH, K)
    dstw = dst.reshape(NW, NCH, K)

    onesb = jnp.ones((K, DEGW), jnp.float32)
    zdeg = jnp.zeros((NP, DEGW), jnp.float32)
    z128 = jnp.zeros((NP, 128), jnp.float32)
    z64 = jnp.zeros((NP, 64), jnp.float32)

    degp = _deg_call(srcw, onesb, zdeg)

    aw2p = jnp.zeros((128, 128), jnp.float32).at[:, :4].set(p['att_W2'])
    ab2p = jnp.full((1, 128), -1e30, jnp.float32).at[0, :4].set(p['att_b2'])
    e4 = jnp.asarray(_E4)

    r = lambda a: a.reshape(1, -1)
    h0, ixm, ixa = pl.pallas_call(
        _b0_body,
        out_shape=[_sds((N, 128)), _sds((N, 64)), _sds((N, 32))],
    )(x, p['att_W1'], r(p['att_b1']), aw2p, ab2p, e4,
      p['fcm_W'], r(p['fcm_b']), r(p['bnm_g']), r(p['bnm_b']),
      p['fca_W'], r(p['fca_b']), r(p['bna_g']), r(p['bna_b']))
    dis, g1 = pl.pallas_call(
        _b1_body,
        out_shape=[_sds((N, 1)), _sds((N, 128))],
    )(h0, degp)

    s1p = _gs_call(g1, srcw, dstw, z128)
    h1, g2 = pl.pallas_call(
        _d1_body,
        out_shape=[_sds((N, 128)), _sds((N, 128))],
    )(h0, s1p, dis, p['c1_W0'], p['c1_W1'], r(p['c1_b']),
      r(p['bn1_g']), r(p['bn1_b']))

    s2p = _gs_call(g2, srcw, dstw, z128)
    h2, g3 = pl.pallas_call(
        _d2_body,
        out_shape=[_sds((N, 64)), _sds((N, 64))],
    )(h1, s2p, dis, p['c2_W0'], p['c2_W1'], r(p['c2_b']),
      r(p['bn2_g']), r(p['bn2_b']), ixm, p['alpha_main'].reshape(1, 1))

    s3p = _gs_call(g3, srcw, dstw, z64)
    out = pl.pallas_call(
        _d3_body,
        out_shape=_sds((N, 32)),
    )(h2, s3p, dis, p['c3_W0'], p['c3_W1'], r(p['c3_b']),
      r(p['bn3_g']), r(p['bn3_b']), ixa, p['alpha_aux'].reshape(1, 1))
    return out
